# baseline (speedup 1.0000x reference)
import functools

import numpy as np
import jax
import jax.numpy as jnp
from jax import lax
from jax.experimental import pallas as pl
from jax.experimental.pallas import tpu as pltpu
from jax.experimental.pallas import tpu_sc as plsc

_MAX_SEQ_LEN = 200
_EMBED_DIM = 64


def _pos_table(max_seq_length, embed_dim):
    pe = np.zeros((max_seq_length, embed_dim), dtype=np.float64)
    pos = np.arange(max_seq_length, dtype=np.float64)[:, None]
    i_even = np.arange(0, embed_dim, 2, dtype=np.float64)
    pe[:, 0::2] = np.sin(pos / np.power(10000.0, i_even / embed_dim))
    pe[:, 1::2] = np.cos(pos / np.power(10000.0, (i_even + 1.0) / embed_dim))
    return pe.astype(np.float32)


_POS_NP = _pos_table(_MAX_SEQ_LEN, _EMBED_DIM)

_NC = 2
_NS = 16
_NW = _NC * _NS
_NBUF = 4


@functools.partial(jax.jit, static_argnames=("batch", "seq"))
def _embed_lookup(idx, table, pos, *, batch, seq):
    d = table.shape[1]
    rows_per_w = batch // _NW

    mesh = plsc.VectorSubcoreMesh(core_axis_name="c", subcore_axis_name="s")

    scratch = [pltpu.VMEM((rows_per_w, seq), jnp.int32)]
    scratch += [pltpu.VMEM((seq, d), jnp.float32) for _ in range(_NBUF)]
    scratch += [pltpu.SemaphoreType.DMA for _ in range(3 * _NBUF + 1)]

    @functools.partial(
        pl.kernel,
        mesh=mesh,
        out_type=jax.ShapeDtypeStruct((batch, seq, d), jnp.float32),
        scratch_types=scratch,
        compiler_params=pltpu.CompilerParams(use_tc_tiling_on_sc=False),
    )
    def _k(idx_hbm, table_hbm, pos_hbm, out_hbm, idx_v, *bufs_and_sems):
        bufs = bufs_and_sems[:_NBUF]
        fsem = bufs_and_sems[_NBUF:2 * _NBUF]
        gsem = bufs_and_sems[2 * _NBUF:3 * _NBUF]
        ssem = bufs_and_sems[3 * _NBUF:4 * _NBUF]
        isem = bufs_and_sems[4 * _NBUF]

        wid = lax.axis_index("s") * _NC + lax.axis_index("c")
        base = wid * rows_per_w

        idx_d = pltpu.async_copy(idx_hbm.at[pl.ds(base, rows_per_w)], idx_v,
                                 isem)

        fill_d, gath_d, store_d = {}, {}, {}

        def fill(c):
            b = c % _NBUF
            if c >= _NBUF:
                store_d[c - _NBUF].wait()
            fill_d[c] = pltpu.async_copy(pos_hbm, bufs[b], fsem[b])

        def gather(c):
            b = c % _NBUF
            fill_d[c].wait()
            gath_d[c] = pltpu.async_copy(
                table_hbm.at[idx_v.at[c]], bufs[b], gsem[b], add=True)

        def store(c):
            b = c % _NBUF
            gath_d[c].wait()
            store_d[c] = pltpu.async_copy(bufs[b], out_hbm.at[base + c],
                                          ssem[b])

        for c in range(_NBUF):
            fill(c)
        idx_d.wait()
        for c in range(min(2, rows_per_w)):
            gather(c)
        for c in range(rows_per_w):
            store(c)
            if c + 2 < rows_per_w:
                gather(c + 2)
            if c + _NBUF < rows_per_w:
                fill(c + _NBUF)
        for c in range(max(0, rows_per_w - _NBUF), rows_per_w):
            store_d[c].wait()

    return _k(idx, table, pos)


def kernel(inputs, token_table):
    batch, seq = inputs.shape
    idx = inputs.astype(jnp.int32)
    pos = jnp.asarray(_POS_NP[:seq])
    return _embed_lookup(idx, token_table, pos, batch=batch, seq=seq)
